# Initial kernel scaffold; baseline (speedup 1.0000x reference)
import jax
import jax.numpy as jnp
from jax.experimental import pallas as pl


def kernel(x, token_table, pos_table):
    raise NotImplementedError("write your pallas kernel here")


# speedup vs baseline: 1.9366x; 1.9366x over previous
import functools

import jax
import jax.numpy as jnp
from jax import lax
from jax.experimental import pallas as pl
from jax.experimental.pallas import tpu as pltpu
from jax.experimental.pallas import tpu_sc as plsc

MAXLEN_ = 200
EMBED_ = 64
BATCH_ = 1024
NWORK_ = 32
CHUNK_ = 128
ROWS_PER_W_ = (BATCH_ * MAXLEN_) // NWORK_
NCHUNK_ = ROWS_PER_W_ // CHUNK_
POS_EXT_ = 336


def _emb_kernel(x_hbm, tok_hbm, pos_hbm, out_hbm, idx_v, pos_v, tok_buf, sem):
    nc = 2
    wid = lax.axis_index("s") * nc + lax.axis_index("c")
    base = wid * ROWS_PER_W_

    pltpu.sync_copy(x_hbm.at[wid], idx_v)
    pltpu.sync_copy(pos_hbm, pos_v)

    def chunk_body(k, carry):
        off = (k * CHUNK_) % MAXLEN_
        pltpu.async_copy(tok_hbm.at[idx_v.at[k]], tok_buf, sem).wait()

        def row_body(i, c):
            for j in range(EMBED_ // 16):
                sl = pl.ds(16 * j, 16)
                tok_buf[i, sl] = tok_buf[i, sl] + pos_v[off + i, sl]
            return c

        lax.fori_loop(0, CHUNK_, row_body, 0)
        pltpu.sync_copy(tok_buf, out_hbm.at[pl.ds(base + k * CHUNK_, CHUNK_)])
        return carry

    lax.fori_loop(0, NCHUNK_, chunk_body, 0)


def kernel(x, token_table, pos_table):
    batch, seqlen = x.shape
    x_view = x.reshape(NWORK_, NCHUNK_, CHUNK_).astype(jnp.int32)
    pos_ext = jnp.concatenate([pos_table, pos_table[: POS_EXT_ - MAXLEN_]], axis=0)

    mesh = plsc.VectorSubcoreMesh(core_axis_name="c", subcore_axis_name="s")
    run = functools.partial(
        pl.kernel,
        mesh=mesh,
        compiler_params=pltpu.CompilerParams(use_tc_tiling_on_sc=False),
        out_type=jax.ShapeDtypeStruct((batch * seqlen, EMBED_), jnp.float32),
        scratch_types=[
            pltpu.VMEM((NCHUNK_, CHUNK_), jnp.int32),
            pltpu.VMEM((POS_EXT_, EMBED_), jnp.float32),
            pltpu.VMEM((CHUNK_, EMBED_), jnp.float32),
            pltpu.SemaphoreType.DMA,
        ],
    )(_emb_kernel)
    out = run(x_view, token_table, pos_ext)
    return out.reshape(batch, seqlen, EMBED_)


# speedup vs baseline: 2.9806x; 1.5391x over previous
import functools

import jax
import jax.numpy as jnp
from jax import lax
from jax.experimental import pallas as pl
from jax.experimental.pallas import tpu as pltpu
from jax.experimental.pallas import tpu_sc as plsc

MAXLEN_ = 200
EMBED_ = 64
BATCH_ = 1024
NWORK_ = 32
CHUNK_ = 128
ROWS_PER_W_ = (BATCH_ * MAXLEN_) // NWORK_
NCHUNK_ = ROWS_PER_W_ // CHUNK_
POS_EXT_ = 336


def _emb_kernel(x_hbm, tok_hbm, pos_hbm, out_hbm, idx_v, pos_v,
                tok0, tok1, gs0, gs1, os0, os1):
    nc = 2
    wid = lax.axis_index("s") * nc + lax.axis_index("c")
    base = wid * ROWS_PER_W_

    pltpu.sync_copy(x_hbm.at[wid], idx_v)
    pltpu.sync_copy(pos_hbm, pos_v)

    bufs = (tok0, tok1)
    gsems = (gs0, gs1)
    osems = (os0, os1)

    def start_gather(k, b):
        pltpu.async_copy(tok_hbm.at[idx_v.at[k]], bufs[b], gsems[b])

    def wait_gather(b):
        pltpu.make_async_copy(tok_hbm.at[idx_v.at[0]], bufs[b], gsems[b]).wait()

    def start_out(k, b):
        pltpu.async_copy(bufs[b], out_hbm.at[pl.ds(base + k * CHUNK_, CHUNK_)],
                         osems[b])

    def wait_out(b):
        pltpu.make_async_copy(bufs[b], out_hbm.at[pl.ds(base, CHUNK_)],
                              osems[b]).wait()

    start_gather(0, 0)

    def super_body(kk, carry):
        for b in (0, 1):
            k = kk * 2 + b
            nb = 1 - b
            wait_gather(b)

            @pl.when(k + 1 < NCHUNK_)
            def _():
                @pl.when(k > 0)
                def _():
                    wait_out(nb)
                start_gather(k + 1, nb)

            off = (k * CHUNK_) % MAXLEN_
            buf = bufs[b]

            @plsc.parallel_loop(0, CHUNK_, unroll=8)
            def add_body(i):
                for j in range(EMBED_ // 16):
                    sl = pl.ds(16 * j, 16)
                    buf[i, sl] = buf[i, sl] + pos_v[off + i, sl]

            start_out(k, b)
        return carry

    lax.fori_loop(0, NCHUNK_ // 2, super_body, 0)
    wait_out(0)
    wait_out(1)


def kernel(x, token_table, pos_table):
    batch, seqlen = x.shape
    x_view = x.reshape(NWORK_, NCHUNK_, CHUNK_).astype(jnp.int32)
    pos_ext = jnp.concatenate([pos_table, pos_table[: POS_EXT_ - MAXLEN_]], axis=0)

    mesh = plsc.VectorSubcoreMesh(core_axis_name="c", subcore_axis_name="s")
    run = functools.partial(
        pl.kernel,
        mesh=mesh,
        compiler_params=pltpu.CompilerParams(use_tc_tiling_on_sc=False),
        out_type=jax.ShapeDtypeStruct((batch * seqlen, EMBED_), jnp.float32),
        scratch_types=[
            pltpu.VMEM((NCHUNK_, CHUNK_), jnp.int32),
            pltpu.VMEM((POS_EXT_, EMBED_), jnp.float32),
            pltpu.VMEM((CHUNK_, EMBED_), jnp.float32),
            pltpu.VMEM((CHUNK_, EMBED_), jnp.float32),
            pltpu.SemaphoreType.DMA,
            pltpu.SemaphoreType.DMA,
            pltpu.SemaphoreType.DMA,
            pltpu.SemaphoreType.DMA,
        ],
    )(_emb_kernel)
    out = run(x_view, token_table, pos_ext)
    return out.reshape(batch, seqlen, EMBED_)
